# baseline (speedup 1.0000x reference)
import jax
import jax.numpy as jnp
from jax import lax
from jax.experimental import pallas as pl
from jax.experimental.pallas import tpu as pltpu

_B, _C, _H, _W = 4, 96, 384, 384
_K = 16
_CB = 32


def _body(win_ref, out_ref):
    jc = pl.program_id(1)
    out_ref[...] = jnp.zeros_like(out_ref)

    @pl.when(jc == 0)
    def _compute():
        a = jnp.abs(win_ref[0])
        m = jnp.max(a, axis=(1, 2), keepdims=True)
        row = lax.broadcasted_iota(jnp.int32, (_C, _K, _K), 1)
        col = lax.broadcasted_iota(jnp.int32, (_C, _K, _K), 2)
        p = row * _K + col
        idx = jnp.min(jnp.where(a == m, p, _K * _K), axis=(1, 2))
        hit = jnp.any(idx[:, None, None] == p[:1], axis=0)
        patch = jnp.where(hit, win_ref[0, 0], 0.0)
        out_ref[0, 0, 0:_K, 0:_K] = patch


def kernel(input_):
    win = input_[:, :, :_K, :_K]
    grid = (_B, _C // _CB)
    return pl.pallas_call(
        _body,
        grid=grid,
        in_specs=[
            pl.BlockSpec((1, _C, _K, _K), lambda b, jc: (b, 0, 0, 0)),
        ],
        out_specs=pl.BlockSpec((1, _CB, _H, _W), lambda b, jc: (b, jc, 0, 0)),
        out_shape=jax.ShapeDtypeStruct((_B, _C, _H, _W), jnp.float32),
        compiler_params=pltpu.CompilerParams(
            dimension_semantics=("parallel", "parallel"),
        ),
    )(win)


# speedup vs baseline: 1.0117x; 1.0117x over previous
import jax
import jax.numpy as jnp
from jax import lax
from jax.experimental import pallas as pl
from jax.experimental.pallas import tpu as pltpu

_B, _C, _H, _W = 4, 96, 384, 384
_K = 16
_CB = 16


def _body(win_ref, out_ref):
    jc = pl.program_id(1)
    out_ref[...] = jnp.zeros_like(out_ref)

    @pl.when(jc == 0)
    def _compute():
        a = jnp.abs(win_ref[0])
        m = jnp.max(a, axis=(1, 2), keepdims=True)
        row = lax.broadcasted_iota(jnp.int32, (_C, _K, _K), 1)
        col = lax.broadcasted_iota(jnp.int32, (_C, _K, _K), 2)
        p = row * _K + col
        idx = jnp.min(jnp.where(a == m, p, _K * _K), axis=(1, 2))
        hit = jnp.any(idx[:, None, None] == p[:1], axis=0)
        patch = jnp.where(hit, win_ref[0, 0], 0.0)
        out_ref[0, 0, 0:_K, 0:_K] = patch


def kernel(input_):
    win = input_[:, :, :_K, :_K]
    grid = (_B, _C // _CB)
    return pl.pallas_call(
        _body,
        grid=grid,
        in_specs=[
            pl.BlockSpec((1, _C, _K, _K), lambda b, jc: (b, 0, 0, 0)),
        ],
        out_specs=pl.BlockSpec((1, _CB, _H, _W), lambda b, jc: (b, jc, 0, 0)),
        out_shape=jax.ShapeDtypeStruct((_B, _C, _H, _W), jnp.float32),
        compiler_params=pltpu.CompilerParams(
            dimension_semantics=("parallel", "parallel"),
        ),
    )(win)


# speedup vs baseline: 1.0596x; 1.0474x over previous
import jax
import jax.numpy as jnp
from jax import lax
from jax.experimental import pallas as pl
from jax.experimental.pallas import tpu as pltpu

_B, _C, _H, _W = 4, 96, 384, 384
_K = 16
_CB = 16


def _body(win_ref, out_ref):
    jc = pl.program_id(1)
    out_ref[...] = jnp.zeros_like(out_ref)

    @pl.when(jc == 0)
    def _compute():
        a = jnp.abs(win_ref[0, :, :, : _K])
        m = jnp.max(a, axis=(1, 2), keepdims=True)
        row = lax.broadcasted_iota(jnp.int32, (_C, _K, _K), 1)
        col = lax.broadcasted_iota(jnp.int32, (_C, _K, _K), 2)
        p = row * _K + col
        idx = jnp.min(jnp.where(a == m, p, _K * _K), axis=(1, 2))
        hit = jnp.any(idx[:, None, None] == p[:1], axis=0)
        patch = jnp.where(hit, win_ref[0, 0, :, : _K], 0.0)
        out_ref[0, 0, 0:_K, 0:_K] = patch


def kernel(input_):
    grid = (_B, _C // _CB)
    return pl.pallas_call(
        _body,
        grid=grid,
        in_specs=[
            pl.BlockSpec((1, _C, _K, 128), lambda b, jc: (b, 0, 0, 0)),
        ],
        out_specs=pl.BlockSpec((1, _CB, _H, _W), lambda b, jc: (b, jc, 0, 0)),
        out_shape=jax.ShapeDtypeStruct((_B, _C, _H, _W), jnp.float32),
        compiler_params=pltpu.CompilerParams(
            dimension_semantics=("parallel", "parallel"),
        ),
    )(input_)


# speedup vs baseline: 1.0602x; 1.0005x over previous
import jax
import jax.numpy as jnp
from jax import lax
from jax.experimental import pallas as pl
from jax.experimental.pallas import tpu as pltpu

_B, _C, _H, _W = 4, 96, 384, 384
_K = 16
_CB = 24


def _body(win_ref, out_ref):
    jc = pl.program_id(1)
    out_ref[...] = jnp.zeros_like(out_ref)

    @pl.when(jc == 0)
    def _compute():
        a = jnp.abs(win_ref[0, :, :, : _K])
        m = jnp.max(a, axis=(1, 2), keepdims=True)
        row = lax.broadcasted_iota(jnp.int32, (_C, _K, _K), 1)
        col = lax.broadcasted_iota(jnp.int32, (_C, _K, _K), 2)
        p = row * _K + col
        idx = jnp.min(jnp.where(a == m, p, _K * _K), axis=(1, 2))
        hit = jnp.any(idx[:, None, None] == p[:1], axis=0)
        patch = jnp.where(hit, win_ref[0, 0, :, : _K], 0.0)
        out_ref[0, 0, 0:_K, 0:_K] = patch


def kernel(input_):
    grid = (_B, _C // _CB)
    return pl.pallas_call(
        _body,
        grid=grid,
        in_specs=[
            pl.BlockSpec((1, _C, _K, 128), lambda b, jc: (b, 0, 0, 0)),
        ],
        out_specs=pl.BlockSpec((1, _CB, _H, _W), lambda b, jc: (b, jc, 0, 0)),
        out_shape=jax.ShapeDtypeStruct((_B, _C, _H, _W), jnp.float32),
        compiler_params=pltpu.CompilerParams(
            dimension_semantics=("parallel", "parallel"),
        ),
    )(input_)
